# baseline (speedup 1.0000x reference)
import functools

import jax
import jax.numpy as jnp
from jax import lax
from jax.experimental import pallas as pl
from jax.experimental.pallas import tpu as pltpu
from jax.experimental.pallas import tpu_sc as plsc

D = 64
VOCAB = 1000000
NW = 32
NC = 2
VT_FULL = VOCAB // 128
VT_TAIL = VOCAB - VT_FULL * 128

_MESH = plsc.VectorSubcoreMesh(core_axis_name="c", subcore_axis_name="s")
_PARAMS = pltpu.CompilerParams(
    use_tc_tiling_on_sc=True, needs_layout_passes=False
)


def _iota16():
  return lax.iota(jnp.int32, 16)


@functools.partial(
    pl.kernel,
    mesh=_MESH,
    out_type=jax.ShapeDtypeStruct((VOCAB // 2, 128), jnp.float32),
    scratch_types=[
        pltpu.VMEM((D, 128), jnp.float32),
        pltpu.VMEM((D, 128), jnp.float32),
    ],
    compiler_params=_PARAMS,
)
def _relayout(table_t, pairs, tin, tout):
  wid = lax.axis_index("s") * NC + lax.axis_index("c")
  n_steps = 244 + (wid < VT_FULL - 244 * NW).astype(jnp.int32)

  def transpose_tile(width):
    for vl in range(width):
      row = jnp.full((16,), vl, jnp.int32)
      for dg in range(4):
        col = _iota16() + dg * 16
        vec = plsc.load_gather(tin, [col, row])
        tout[vl // 2, pl.ds((vl % 2) * 64 + dg * 16, 16)] = vec

  def step(i, carry):
    vt = i * NW + wid
    pltpu.sync_copy(table_t.at[:, pl.ds(vt * 128, 128)], tin)
    transpose_tile(128)
    pltpu.sync_copy(tout, pairs.at[pl.ds(vt * 64, 64), :])
    return carry

  lax.fori_loop(0, n_steps, step, 0)


@functools.partial(
    pl.kernel,
    mesh=_MESH,
    out_type=jax.ShapeDtypeStruct((200, D, 4096), jnp.float32),
    scratch_types=[
        pltpu.VMEM((128,), jnp.int32),
        pltpu.VMEM((128,), jnp.int32),
        pltpu.VMEM((128, 128), jnp.float32),
        pltpu.VMEM((D, 128), jnp.float32),
        pltpu.SemaphoreType.DMA,
    ],
    compiler_params=_PARAMS,
)
def _gather(x_t, pairs, out, idx_v, gidx, prows, tout, sem):
  wid = lax.axis_index("s") * NC + lax.axis_index("c")

  def step(t, carry):
    pltpu.sync_copy(x_t.at[t, pl.ds(wid * 128, 128)], idx_v)
    for k in range(8):
      v = idx_v[pl.ds(k * 16, 16)]
      gidx[pl.ds(k * 16, 16)] = lax.shift_right_logical(v, 1)
      idx_v[pl.ds(k * 16, 16)] = lax.shift_left(jnp.bitwise_and(v, 1), 6)
    pltpu.async_copy(pairs.at[gidx], prows, sem).wait()
    for bg in range(8):
      rows = _iota16() + bg * 16
      half = idx_v[pl.ds(bg * 16, 16)]
      for d in range(D):
        vec = plsc.load_gather(prows, [rows, half + d])
        tout[d, pl.ds(bg * 16, 16)] = vec
    pltpu.sync_copy(tout, out.at[t, :, pl.ds(wid * 128, 128)])
    return carry

  lax.fori_loop(0, 200, step, 0)


@jax.jit
def kernel(x, table):
  B0, T = x.shape
  xt = jnp.asarray(x, jnp.int32).T
  tt = table.T
  pairs = _relayout(tt)
  tail = table[VT_FULL * 128 :, :].reshape(VT_TAIL // 2, 128)
  pairs = lax.dynamic_update_slice(pairs, tail, (VT_FULL * 64, 0))
  out5 = _gather(xt, pairs)
  return out5.transpose(2, 0, 1)


# speedup vs baseline: 1.3710x; 1.3710x over previous
import functools

import jax
import jax.numpy as jnp
from jax import lax
from jax.experimental import pallas as pl
from jax.experimental.pallas import tpu as pltpu
from jax.experimental.pallas import tpu_sc as plsc

D = 64
VOCAB = 1000000
NW = 32
NC = 2
VT_FULL = VOCAB // 128
VT_TAIL = VOCAB - VT_FULL * 128

_MESH = plsc.VectorSubcoreMesh(core_axis_name="c", subcore_axis_name="s")
_PARAMS = pltpu.CompilerParams(
    use_tc_tiling_on_sc=True, needs_layout_passes=False
)


def _iota16():
  return lax.iota(jnp.int32, 16)


@functools.partial(
    pl.kernel,
    mesh=_MESH,
    out_type=jax.ShapeDtypeStruct((VOCAB // 2, 128), jnp.float32),
    scratch_types=[
        pltpu.VMEM((D, 128), jnp.float32),
        pltpu.VMEM((D, 128), jnp.float32),
    ],
    compiler_params=_PARAMS,
)
def _relayout(table_t, pairs, tin, tout):
  wid = lax.axis_index("s") * NC + lax.axis_index("c")
  n_steps = 244 + (wid < VT_FULL - 244 * NW).astype(jnp.int32)

  def transpose_tile(width):
    @plsc.parallel_loop(0, width * 4, step=1, unroll=8)
    def _(k):
      vl = lax.shift_right_logical(k, 2)
      dg = jnp.bitwise_and(k, 3)
      row = jnp.broadcast_to(vl, (16,)).astype(jnp.int32)
      col = _iota16() + dg * 16
      vec = plsc.load_gather(tin, [col, row])
      half = lax.shift_right_logical(vl, 1)
      off = jnp.bitwise_and(vl, 1) * 64 + dg * 16
      tout[half, pl.ds(off, 16)] = vec

  def step(i, carry):
    vt = i * NW + wid
    pltpu.sync_copy(table_t.at[:, pl.ds(vt * 128, 128)], tin)
    transpose_tile(128)
    pltpu.sync_copy(tout, pairs.at[pl.ds(vt * 64, 64), :])
    return carry

  lax.fori_loop(0, n_steps, step, 0)


@functools.partial(
    pl.kernel,
    mesh=_MESH,
    out_type=jax.ShapeDtypeStruct((200, D, 4096), jnp.float32),
    scratch_types=[
        pltpu.VMEM((128,), jnp.int32),
        pltpu.VMEM((128,), jnp.int32),
        pltpu.VMEM((128, 128), jnp.float32),
        pltpu.VMEM((D, 128), jnp.float32),
        pltpu.SemaphoreType.DMA,
    ],
    compiler_params=_PARAMS,
)
def _gather(x_t, pairs, out, idx_v, gidx, prows, tout, sem):
  wid = lax.axis_index("s") * NC + lax.axis_index("c")

  def step(t, carry):
    pltpu.sync_copy(x_t.at[t, pl.ds(wid * 128, 128)], idx_v)
    for k in range(8):
      v = idx_v[pl.ds(k * 16, 16)]
      gidx[pl.ds(k * 16, 16)] = lax.shift_right_logical(v, 1)
      idx_v[pl.ds(k * 16, 16)] = lax.shift_left(jnp.bitwise_and(v, 1), 6)
    pltpu.async_copy(pairs.at[gidx], prows, sem).wait()
    @plsc.parallel_loop(0, 8 * D, step=1, unroll=8)
    def _(k):
      bg = lax.shift_right_logical(k, 6)
      d = jnp.bitwise_and(k, D - 1)
      rows = _iota16() + bg * 16
      half = idx_v[pl.ds(bg * 16, 16)]
      vec = plsc.load_gather(prows, [rows, half + d])
      tout[d, pl.ds(bg * 16, 16)] = vec
    pltpu.sync_copy(tout, out.at[t, :, pl.ds(wid * 128, 128)])
    return carry

  lax.fori_loop(0, 200, step, 0)


@jax.jit
def kernel(x, table):
  B0, T = x.shape
  xt = jnp.asarray(x, jnp.int32).T
  tt = table.T
  pairs = _relayout(tt)
  tail = table[VT_FULL * 128 :, :].reshape(VT_TAIL // 2, 128)
  pairs = lax.dynamic_update_slice(pairs, tail, (VT_FULL * 64, 0))
  out5 = _gather(xt, pairs)
  return out5.transpose(2, 0, 1)


# speedup vs baseline: 1.5753x; 1.1490x over previous
import functools

import jax
import jax.numpy as jnp
from jax import lax
from jax.experimental import pallas as pl
from jax.experimental.pallas import tpu as pltpu
from jax.experimental.pallas import tpu_sc as plsc

D = 64
VOCAB = 1000000
NW = 32
NC = 2
VT_FULL = VOCAB // 128
VT_TAIL = VOCAB - VT_FULL * 128

_MESH = plsc.VectorSubcoreMesh(core_axis_name="c", subcore_axis_name="s")
_PARAMS = pltpu.CompilerParams(
    use_tc_tiling_on_sc=True, needs_layout_passes=False
)


def _iota16():
  return lax.iota(jnp.int32, 16)


@functools.partial(
    pl.kernel,
    mesh=_MESH,
    out_type=jax.ShapeDtypeStruct((VOCAB // 2, 128), jnp.float32),
    scratch_types=[
        pltpu.VMEM((D, 128), jnp.float32),
        pltpu.VMEM((D, 128), jnp.float32),
    ],
    compiler_params=_PARAMS,
)
def _relayout(table_t, pairs, tin, tout):
  wid = lax.axis_index("s") * NC + lax.axis_index("c")
  n_steps = 244 + (wid < VT_FULL - 244 * NW).astype(jnp.int32)

  cols = [_iota16() + dg * 16 for dg in range(4)]

  def transpose_tile(width):
    @plsc.parallel_loop(0, width, step=1, unroll=8)
    def _(vl):
      row = jnp.broadcast_to(vl, (16,)).astype(jnp.int32)
      half = lax.shift_right_logical(vl, 1)
      off0 = jnp.bitwise_and(vl, 1) * 64
      for dg in range(4):
        vec = plsc.load_gather(tin, [cols[dg], row])
        tout[half, pl.ds(off0 + dg * 16, 16)] = vec

  def step(i, carry):
    vt = i * NW + wid
    pltpu.sync_copy(table_t.at[:, pl.ds(vt * 128, 128)], tin)
    transpose_tile(128)
    pltpu.sync_copy(tout, pairs.at[pl.ds(vt * 64, 64), :])
    return carry

  lax.fori_loop(0, n_steps, step, 0)


@functools.partial(
    pl.kernel,
    mesh=_MESH,
    out_type=jax.ShapeDtypeStruct((200, D, 4096), jnp.float32),
    scratch_types=[
        pltpu.VMEM((128,), jnp.int32),
        pltpu.VMEM((128,), jnp.int32),
        pltpu.VMEM((128, 128), jnp.float32),
        pltpu.VMEM((D, 128), jnp.float32),
        pltpu.SemaphoreType.DMA,
    ],
    compiler_params=_PARAMS,
)
def _gather(x_t, pairs, out, idx_v, gidx, prows, tout, sem):
  wid = lax.axis_index("s") * NC + lax.axis_index("c")
  rows_l = [_iota16() + bg * 16 for bg in range(8)]

  def step(t, carry):
    pltpu.sync_copy(x_t.at[t, pl.ds(wid * 128, 128)], idx_v)
    for k in range(8):
      v = idx_v[pl.ds(k * 16, 16)]
      gidx[pl.ds(k * 16, 16)] = lax.shift_right_logical(v, 1)
      idx_v[pl.ds(k * 16, 16)] = lax.shift_left(jnp.bitwise_and(v, 1), 6)
    pltpu.async_copy(pairs.at[gidx], prows, sem).wait()
    half_l = [idx_v[pl.ds(bg * 16, 16)] for bg in range(8)]

    @plsc.parallel_loop(0, D, step=1, unroll=4)
    def _(d):
      for bg in range(8):
        vec = plsc.load_gather(prows, [rows_l[bg], half_l[bg] + d])
        tout[d, pl.ds(bg * 16, 16)] = vec
    pltpu.sync_copy(tout, out.at[t, :, pl.ds(wid * 128, 128)])
    return carry

  lax.fori_loop(0, 200, step, 0)


@jax.jit
def kernel(x, table):
  B0, T = x.shape
  xt = jnp.asarray(x, jnp.int32).T
  tt = table.T
  pairs = _relayout(tt)
  tail = table[VT_FULL * 128 :, :].reshape(VT_TAIL // 2, 128)
  pairs = lax.dynamic_update_slice(pairs, tail, (VT_FULL * 64, 0))
  out5 = _gather(xt, pairs)
  return out5.transpose(2, 0, 1)


# speedup vs baseline: 1.9297x; 1.2249x over previous
import functools

import jax
import jax.numpy as jnp
from jax import lax
from jax.experimental import pallas as pl
from jax.experimental.pallas import tpu as pltpu
from jax.experimental.pallas import tpu_sc as plsc

D = 64
VOCAB = 1000000
NW = 32
NC = 2
VT_FULL = VOCAB // 128
VT_TAIL = VOCAB - VT_FULL * 128

_MESH = plsc.VectorSubcoreMesh(core_axis_name="c", subcore_axis_name="s")
_PARAMS = pltpu.CompilerParams(
    use_tc_tiling_on_sc=True, needs_layout_passes=False
)


def _iota16():
  return lax.iota(jnp.int32, 16)


@functools.partial(
    pl.kernel,
    mesh=_MESH,
    out_type=jax.ShapeDtypeStruct((VOCAB // 2, 128), jnp.float32),
    scratch_types=[
        pltpu.VMEM((D, 128), jnp.float32),
        pltpu.VMEM((D, 128), jnp.float32),
    ],
    compiler_params=_PARAMS,
)
def _relayout(table_t, pairs, tin, tout):
  wid = lax.axis_index("s") * NC + lax.axis_index("c")
  n_steps = 244 + (wid < VT_FULL - 244 * NW).astype(jnp.int32)

  rowd = [_iota16() + dg * 16 for dg in range(4)]
  perms = [jnp.bitwise_and(_iota16() + k, 15) for k in range(16)]
  cpart = [
      lax.shift_left(jnp.bitwise_and(p, 1), 6) + _iota16() for p in perms
  ]

  def transpose_tile():
    @plsc.parallel_loop(0, 8, step=1, unroll=2)
    def _(vg):
      vl0 = vg * 16
      for k in range(16):
        colv = perms[k] + vl0
        rv = lax.shift_right_logical(colv, 1)
        for dg in range(4):
          vec = plsc.load_gather(tin, [rowd[dg], colv])
          cv = cpart[k] + dg * 16
          plsc.store_scatter(tout, [rv, cv], vec)

  def step(i, carry):
    vt = i * NW + wid
    pltpu.sync_copy(table_t.at[:, pl.ds(vt * 128, 128)], tin)
    transpose_tile()
    pltpu.sync_copy(tout, pairs.at[pl.ds(vt * 64, 64), :])
    return carry

  lax.fori_loop(0, n_steps, step, 0)


@functools.partial(
    pl.kernel,
    mesh=_MESH,
    out_type=jax.ShapeDtypeStruct((200, D, 4096), jnp.float32),
    scratch_types=[
        pltpu.VMEM((128,), jnp.int32),
        pltpu.VMEM((128,), jnp.int32),
        pltpu.VMEM((128, 128), jnp.float32),
        pltpu.VMEM((D, 128), jnp.float32),
        pltpu.SemaphoreType.DMA,
    ],
    compiler_params=_PARAMS,
)
def _gather(x_t, pairs, out, idx_v, gidx, prows, tout, sem):
  wid = lax.axis_index("s") * NC + lax.axis_index("c")
  rowd = [_iota16() + dg * 16 for dg in range(4)]
  perms = [jnp.bitwise_and(_iota16() + k, 15) for k in range(16)]

  def step(t, carry):
    pltpu.sync_copy(x_t.at[t, pl.ds(wid * 128, 128)], idx_v)
    for k in range(8):
      v = idx_v[pl.ds(k * 16, 16)]
      gidx[pl.ds(k * 16, 16)] = lax.shift_right_logical(v, 1)
      idx_v[pl.ds(k * 16, 16)] = lax.shift_left(jnp.bitwise_and(v, 1), 6)
    pltpu.async_copy(pairs.at[gidx], prows, sem).wait()
    @plsc.parallel_loop(0, 8, step=1, unroll=2)
    def _(bg):
      b0 = bg * 16
      for k in range(16):
        cold = perms[k] + b0
        halfp = plsc.load_gather(idx_v, [cold])
        for dg in range(4):
          colv = halfp + rowd[dg]
          vec = plsc.load_gather(prows, [cold, colv])
          plsc.store_scatter(tout, [rowd[dg], cold], vec)
    pltpu.sync_copy(tout, out.at[t, :, pl.ds(wid * 128, 128)])
    return carry

  lax.fori_loop(0, 200, step, 0)


@jax.jit
def kernel(x, table):
  B0, T = x.shape
  xt = jnp.asarray(x, jnp.int32).T
  tt = table.T
  pairs = _relayout(tt)
  tail = table[VT_FULL * 128 :, :].reshape(VT_TAIL // 2, 128)
  pairs = lax.dynamic_update_slice(pairs, tail, (VT_FULL * 64, 0))
  out5 = _gather(xt, pairs)
  return out5.transpose(2, 0, 1)


# speedup vs baseline: 2.9149x; 1.5106x over previous
import functools

import jax
import jax.numpy as jnp
from jax import lax
from jax.experimental import pallas as pl
from jax.experimental.pallas import tpu as pltpu
from jax.experimental.pallas import tpu_sc as plsc

D = 64
VOCAB = 1000000
NW = 32
NC = 2
VT_FULL = VOCAB // 128
VT_TAIL = VOCAB - VT_FULL * 128

_MESH = plsc.VectorSubcoreMesh(core_axis_name="c", subcore_axis_name="s")
_PARAMS = pltpu.CompilerParams(
    use_tc_tiling_on_sc=True, needs_layout_passes=False
)


def _iota16():
  return lax.iota(jnp.int32, 16)


@functools.partial(
    pl.kernel,
    mesh=_MESH,
    out_type=jax.ShapeDtypeStruct((VOCAB // 2, 128), jnp.float32),
    scratch_types=[
        pltpu.VMEM((D, 128), jnp.float32),
        pltpu.VMEM((D, 128), jnp.float32),
    ],
    compiler_params=_PARAMS,
)
def _relayout(table_t, pairs, tin, tout):
  wid = lax.axis_index("s") * NC + lax.axis_index("c")
  n_steps = 244 + (wid < VT_FULL - 244 * NW).astype(jnp.int32)

  rowd = [_iota16() + dg * 16 for dg in range(4)]
  perms = [jnp.bitwise_and(_iota16() + k, 15) for k in range(16)]
  cpart = [
      lax.shift_left(jnp.bitwise_and(p, 1), 6) + _iota16() for p in perms
  ]

  def transpose_tile():
    @plsc.parallel_loop(0, 8, step=1, unroll=4)
    def _(vg):
      vl0 = vg * 16
      for k in range(16):
        colv = perms[k] + vl0
        rv = lax.shift_right_logical(colv, 1)
        vecs = [plsc.load_gather(tin, [rowd[dg], colv]) for dg in range(4)]
        for dg in range(4):
          cv = cpart[k] + dg * 16
          plsc.store_scatter(tout, [rv, cv], vecs[dg])

  def step(i, carry):
    vt = i * NW + wid
    pltpu.sync_copy(table_t.at[:, pl.ds(vt * 128, 128)], tin)
    transpose_tile()
    pltpu.sync_copy(tout, pairs.at[pl.ds(vt * 64, 64), :])
    return carry

  lax.fori_loop(0, n_steps, step, 0)


@functools.partial(
    pl.kernel,
    mesh=_MESH,
    out_type=jax.ShapeDtypeStruct((200, D, 4096), jnp.float32),
    scratch_types=[
        pltpu.VMEM((128,), jnp.int32),
        pltpu.VMEM((128,), jnp.int32),
        pltpu.VMEM((128, 128), jnp.float32),
        pltpu.VMEM((D, 128), jnp.float32),
        pltpu.SemaphoreType.DMA,
    ],
    compiler_params=_PARAMS,
)
def _gather(x_t, pairs, out, idx_v, gidx, prows, tout, sem):
  wid = lax.axis_index("s") * NC + lax.axis_index("c")
  rowd = [_iota16() + dg * 16 for dg in range(4)]
  perms = [jnp.bitwise_and(_iota16() + k, 15) for k in range(16)]

  def step(t, carry):
    pltpu.sync_copy(x_t.at[t, pl.ds(wid * 128, 128)], idx_v)
    for k in range(8):
      v = idx_v[pl.ds(k * 16, 16)]
      gidx[pl.ds(k * 16, 16)] = lax.shift_right_logical(v, 1)
      idx_v[pl.ds(k * 16, 16)] = lax.shift_left(jnp.bitwise_and(v, 1), 6)
    pltpu.async_copy(pairs.at[gidx], prows, sem).wait()
    @plsc.parallel_loop(0, 8, step=1, unroll=4)
    def _(bg):
      b0 = bg * 16
      for k in range(16):
        cold = perms[k] + b0
        halfp = plsc.load_gather(idx_v, [cold])
        vecs = [
            plsc.load_gather(prows, [cold, halfp + rowd[dg]])
            for dg in range(4)
        ]
        for dg in range(4):
          plsc.store_scatter(tout, [rowd[dg], cold], vecs[dg])
    pltpu.sync_copy(tout, out.at[t, :, pl.ds(wid * 128, 128)])
    return carry

  lax.fori_loop(0, 200, step, 0)


@jax.jit
def kernel(x, table):
  B0, T = x.shape
  xt = jnp.asarray(x, jnp.int32).T
  tt = table.T
  pairs = _relayout(tt)
  tail = table[VT_FULL * 128 :, :].reshape(VT_TAIL // 2, 128)
  pairs = lax.dynamic_update_slice(pairs, tail, (VT_FULL * 64, 0))
  out5 = _gather(xt, pairs)
  return out5.transpose(2, 0, 1)


# speedup vs baseline: 5.6882x; 1.9515x over previous
import functools

import jax
import jax.numpy as jnp
from jax import lax
from jax.experimental import pallas as pl
from jax.experimental.pallas import tpu as pltpu
from jax.experimental.pallas import tpu_sc as plsc

D = 64
VOCAB = 1000000
NW = 32
NC = 2
VT_FULL = VOCAB // 128
VT_TAIL = VOCAB - VT_FULL * 128
NT = 200

_MESH = plsc.VectorSubcoreMesh(core_axis_name="c", subcore_axis_name="s")
_PARAMS = pltpu.CompilerParams(
    use_tc_tiling_on_sc=True, needs_layout_passes=False
)


def _iota16():
  return lax.iota(jnp.int32, 16)


@functools.partial(
    pl.kernel,
    mesh=_MESH,
    out_type=jax.ShapeDtypeStruct((VOCAB // 2, 128), jnp.float32),
    scratch_types=[
        pltpu.VMEM((2, D, 128), jnp.float32),
        pltpu.VMEM((2, D, 128), jnp.float32),
        pltpu.SemaphoreType.DMA,
        pltpu.SemaphoreType.DMA,
        pltpu.SemaphoreType.DMA,
        pltpu.SemaphoreType.DMA,
    ],
    compiler_params=_PARAMS,
)
def _relayout(table_t, pairs, tin, tout, gi0, gi1, go0, go1):
  wid = lax.axis_index("s") * NC + lax.axis_index("c")
  n_steps = 244 + (wid < VT_FULL - 244 * NW).astype(jnp.int32)
  isems = [gi0, gi1]
  osems = [go0, go1]

  rowd = [_iota16() + dg * 16 for dg in range(4)]
  perms = [jnp.bitwise_and(_iota16() + k, 15) for k in range(16)]
  cpart = [
      lax.shift_left(jnp.bitwise_and(p, 1), 6) + _iota16() for p in perms
  ]

  def start_in(i, half):
    vt = i * NW + wid
    pltpu.async_copy(
        table_t.at[:, pl.ds(vt * 128, 128)], tin.at[half], isems[half]
    )

  def drain_in(half):
    pltpu.make_async_copy(
        table_t.at[:, pl.ds(0, 128)], tin.at[half], isems[half]
    ).wait()

  def drain_out(half):
    pltpu.make_async_copy(
        tout.at[half], pairs.at[pl.ds(0, 64), :], osems[half]
    ).wait()

  def transpose_tile(half):
    src = tin.at[half]
    dst = tout.at[half]

    @plsc.parallel_loop(0, 8, step=1, unroll=4)
    def _(vg):
      vl0 = vg * 16
      for k in range(16):
        colv = perms[k] + vl0
        rv = lax.shift_right_logical(colv, 1)
        vecs = [plsc.load_gather(src, [rowd[dg], colv]) for dg in range(4)]
        for dg in range(4):
          cv = cpart[k] + dg * 16
          plsc.store_scatter(dst, [rv, cv], vecs[dg])

  start_in(0, 0)
  start_in(1, 1)

  def pair_step(p, carry):
    for half in range(2):
      i = 2 * p + half

      @pl.when(i < n_steps)
      def _():
        drain_in(half)

        @pl.when(i >= 2)
        def _():
          drain_out(half)

        transpose_tile(half)
        vt = i * NW + wid
        pltpu.async_copy(
            tout.at[half], pairs.at[pl.ds(vt * 64, 64), :], osems[half]
        )

        @pl.when(i + 2 < n_steps)
        def _():
          start_in(i + 2, half)

    return carry

  lax.fori_loop(0, 123, pair_step, 0)
  drain_out(0)
  drain_out(1)


@functools.partial(
    pl.kernel,
    mesh=_MESH,
    out_type=jax.ShapeDtypeStruct((NT, D, 4096), jnp.float32),
    scratch_types=[
        pltpu.VMEM((NT, 128), jnp.int32),
        pltpu.VMEM((2, 128), jnp.int32),
        pltpu.VMEM((2, 128), jnp.int32),
        pltpu.VMEM((2, 128, 128), jnp.float32),
        pltpu.VMEM((2, D, 128), jnp.float32),
        pltpu.SemaphoreType.DMA,
        pltpu.SemaphoreType.DMA,
        pltpu.SemaphoreType.DMA,
        pltpu.SemaphoreType.DMA,
    ],
    compiler_params=_PARAMS,
)
def _gather(x_t, pairs, out, idx_all, gidx, hbuf, prows, tout, gg0, gg1, go0,
            go1):
  wid = lax.axis_index("s") * NC + lax.axis_index("c")
  gsems = [gg0, gg1]
  osems = [go0, go1]
  rowd = [_iota16() + dg * 16 for dg in range(4)]
  perms = [jnp.bitwise_and(_iota16() + k, 15) for k in range(16)]

  pltpu.sync_copy(x_t.at[:, pl.ds(wid * 128, 128)], idx_all)

  def prep_and_fire(t, buf):
    for k in range(8):
      v = idx_all[t, pl.ds(k * 16, 16)]
      gidx[buf, pl.ds(k * 16, 16)] = lax.shift_right_logical(v, 1)
      hbuf[buf, pl.ds(k * 16, 16)] = lax.shift_left(
          jnp.bitwise_and(v, 1), 6
      )
    pltpu.async_copy(pairs.at[gidx.at[buf]], prows.at[buf], gsems[buf])

  def drain_gather(buf):
    pltpu.make_async_copy(
        pairs.at[gidx.at[buf]], prows.at[buf], gsems[buf]
    ).wait()

  def drain_out(buf):
    pltpu.make_async_copy(
        tout.at[buf], out.at[0, :, pl.ds(0, 128)], osems[buf]
    ).wait()

  def transpose_block(buf):
    src = prows.at[buf]
    dst = tout.at[buf]
    hsrc = hbuf.at[buf]

    @plsc.parallel_loop(0, 8, step=1, unroll=4)
    def _(bg):
      b0 = bg * 16
      for k in range(16):
        cold = perms[k] + b0
        halfp = plsc.load_gather(hsrc, [cold])
        vecs = [
            plsc.load_gather(src, [cold, halfp + rowd[dg]])
            for dg in range(4)
        ]
        for dg in range(4):
          plsc.store_scatter(dst, [rowd[dg], cold], vecs[dg])

  prep_and_fire(0, 0)

  def pair_step(p, carry):
    for half in range(2):
      i = 2 * p + half

      @pl.when(i + 1 < NT)
      def _():
        prep_and_fire(i + 1, 1 - half)

      drain_gather(half)

      @pl.when(i >= 2)
      def _():
        drain_out(half)

      transpose_block(half)
      pltpu.async_copy(
          tout.at[half], out.at[i, :, pl.ds(wid * 128, 128)], osems[half]
      )
    return carry

  lax.fori_loop(0, NT // 2, pair_step, 0)
  drain_out(0)
  drain_out(1)


@jax.jit
def kernel(x, table):
  B0, T = x.shape
  xt = jnp.asarray(x, jnp.int32).T
  tt = table.T
  pairs = _relayout(tt)
  tail = table[VT_FULL * 128 :, :].reshape(VT_TAIL // 2, 128)
  pairs = lax.dynamic_update_slice(pairs, tail, (VT_FULL * 64, 0))
  out5 = _gather(xt, pairs)
  return out5.transpose(2, 0, 1)
